# Initial kernel scaffold; baseline (speedup 1.0000x reference)
import jax
import jax.numpy as jnp
from jax.experimental import pallas as pl


def kernel(x, A, mask, k, W_mlp, b_mlp, alpha1, alpha2):
    raise NotImplementedError("write your pallas kernel here")


# speedup vs baseline: 20.4408x; 20.4408x over previous
import functools
import math

import jax
import jax.numpy as jnp
from jax.experimental import pallas as pl

_B, _C1, _C, _N, _T = 8, 8, 32, 500, 24
_O = 10
_K = 5
_INV_SQRT10 = 1.0 / math.sqrt(10.0)


def _nconv_body(xt_ref, A_ref, mask_ref, W_ref, b_ref, a1_ref, a2_ref,
                xo_ref, mo_ref):
    N = _N
    Af = A_ref[...]
    rows = jax.lax.broadcasted_iota(jnp.int32, (N, N), 0)
    cols = jax.lax.broadcasted_iota(jnp.int32, (N, N), 1)
    alpha_term = 0.001 * (
        jnp.where(cols > rows, a1_ref[...], 0.0)
        + jnp.where(cols <= rows, a2_ref[...], 0.0)
    )
    bias = b_ref[0]
    W = W_ref[...]

    acc = jnp.zeros((N, N), jnp.float32)
    for c1 in range(_C1):
        m = mask_ref[0, c1]
        P = jax.lax.dot_general(m, W, (((1,), (1,)), ((), ())),
                                preferred_element_type=jnp.float32) + bias
        S = jax.lax.dot_general(P, P, (((1,), (1,)), ((), ())),
                                preferred_element_type=jnp.float32)
        mw = jax.nn.sigmoid((S + alpha_term) * _INV_SQRT10)
        A4 = Af + 0.002 * mw
        acc = acc + A4

        a = A4
        out = None
        for j in range(_K):
            mx = jnp.max(a, axis=1, keepdims=True)
            eq = a >= mx
            first = jnp.min(jnp.where(eq, cols, N), axis=1, keepdims=True)
            onehot = cols == first
            sel = jnp.dot(onehot.astype(jnp.float32), m,
                          preferred_element_type=jnp.float32)
            out = sel if out is None else jnp.maximum(out, sel)
            if j + 1 < _K:
                a = jnp.where(onehot, -jnp.inf, a)
        mo_ref[0, c1] = out

    xo_ref[0] = jnp.dot(acc, xt_ref[0], preferred_element_type=jnp.float32)


@jax.jit
def kernel(x, A, mask, k, W_mlp, b_mlp, alpha1, alpha2):
    B, C, N, T = x.shape
    C1 = mask.shape[1]
    O = W_mlp.shape[0]
    x_t = jnp.transpose(x, (0, 2, 1, 3)).reshape(B, N, C * T)
    b2 = b_mlp.reshape(1, O)

    x_out_t, mask_out = pl.pallas_call(
        _nconv_body,
        grid=(B,),
        in_specs=[
            pl.BlockSpec((1, N, C * T), lambda b: (b, 0, 0)),
            pl.BlockSpec((N, N), lambda b: (0, 0)),
            pl.BlockSpec((1, C1, N, T), lambda b: (b, 0, 0, 0)),
            pl.BlockSpec((O, T), lambda b: (0, 0)),
            pl.BlockSpec((1, O), lambda b: (0, 0)),
            pl.BlockSpec((N, N), lambda b: (0, 0)),
            pl.BlockSpec((N, N), lambda b: (0, 0)),
        ],
        out_specs=[
            pl.BlockSpec((1, N, C * T), lambda b: (b, 0, 0)),
            pl.BlockSpec((1, C1, N, T), lambda b: (b, 0, 0, 0)),
        ],
        out_shape=[
            jax.ShapeDtypeStruct((B, N, C * T), jnp.float32),
            jax.ShapeDtypeStruct((B, C1, N, T), jnp.float32),
        ],
    )(x_t, A, mask, W_mlp, b2, alpha1, alpha2)

    x_out = x_out_t.reshape(B, N, C, T).transpose(0, 2, 1, 3)
    return x_out, mask_out
